# baseline (speedup 1.0000x reference)
import jax
import jax.numpy as jnp
from jax import lax
from jax.experimental import pallas as pl
from jax.experimental.pallas import tpu as pltpu
from jax.experimental.pallas import tpu_sc as plsc

B = 16384
N_TOKENS = 43
VOCAB = 1000
DIM = 128

ROWS = B * N_TOKENS
NW = 32
CHUNK = 128
ROWS_PER_W = ROWS // NW
CHUNKS_PER_W = ROWS_PER_W // CHUNK


def _bias_body(id_ref, type_ref, out_ref):
    out_ref[...] = id_ref[0] + type_ref[0]


def _make_bias(id_emb, type_emb):
    return pl.pallas_call(
        _bias_body,
        out_shape=jax.ShapeDtypeStruct((N_TOKENS, DIM), jnp.float32),
    )(id_emb, type_emb)


NBUF = 4
GROUPS = CHUNKS_PER_W // NBUF


def _gather_body(table_hbm, idx_hbm, bias_hbm, out_hbm, idx_v, bias_v,
                 *scratch):
    bufs = scratch[:NBUF]
    gsem = scratch[NBUF:2 * NBUF]
    osem = scratch[2 * NBUF:]
    wid = lax.axis_index("s") * 2 + lax.axis_index("c")
    row_base = wid * ROWS_PER_W
    pltpu.sync_copy(idx_hbm.at[pl.ds(row_base, ROWS_PER_W)], idx_v)
    pltpu.sync_copy(bias_hbm, bias_v)

    def chunk_t(j):
        return (row_base + j * CHUNK) // B

    def tbl_at(j):
        return table_hbm.at[pl.ds(chunk_t(j) * VOCAB, VOCAB)]

    def idx_at(j):
        return idx_v.at[pl.ds(j * CHUNK, CHUNK)]

    def add_bias(j, buf):
        t = chunk_t(j)
        bv = [bias_v[t, pl.ds(16 * k, 16)] for k in range(DIM // 16)]

        def row_add(r, carry):
            for u in range(4):
                for k in range(DIM // 16):
                    sl = pl.ds(16 * k, 16)
                    buf[r + u, sl] = buf[r + u, sl] + bv[k]
            return carry

        lax.fori_loop(0, CHUNK // 4, lambda i, c: row_add(4 * i, c), 0,
                      unroll=False)

    for s in range(NBUF):
        pltpu.async_copy(tbl_at(s).at[idx_at(s)], bufs[s], gsem[s])

    def step(i, carry):
        for s in range(NBUF):
            j = i * NBUF + s
            pltpu.make_async_copy(
                tbl_at(j).at[idx_at(j)], bufs[s], gsem[s]).wait()
            add_bias(j, bufs[s])
            pltpu.async_copy(
                bufs[s], out_hbm.at[pl.ds(row_base + j * CHUNK, CHUNK)],
                osem[s])
        for s in range(NBUF):
            j = i * NBUF + s
            jn = j + NBUF
            pltpu.make_async_copy(
                bufs[s], out_hbm.at[pl.ds(row_base + j * CHUNK, CHUNK)],
                osem[s]).wait()
            pltpu.async_copy(tbl_at(jn).at[idx_at(jn)], bufs[s], gsem[s])
        return carry

    lax.fori_loop(0, GROUPS - 1, step, 0)

    for s in range(NBUF):
        j = (GROUPS - 1) * NBUF + s
        pltpu.make_async_copy(
            tbl_at(j).at[idx_at(j)], bufs[s], gsem[s]).wait()
        add_bias(j, bufs[s])
        pltpu.async_copy(
            bufs[s], out_hbm.at[pl.ds(row_base + j * CHUNK, CHUNK)], osem[s])
    for s in range(NBUF):
        j = (GROUPS - 1) * NBUF + s
        pltpu.make_async_copy(
            bufs[s], out_hbm.at[pl.ds(row_base + j * CHUNK, CHUNK)],
            osem[s]).wait()


def _sc_gather(table, idx_flat, bias):
    mesh = plsc.VectorSubcoreMesh(core_axis_name="c", subcore_axis_name="s")
    run = pl.kernel(
        _gather_body,
        out_type=jax.ShapeDtypeStruct((ROWS, DIM), jnp.float32),
        mesh=mesh,
        scratch_types=[
            pltpu.VMEM((ROWS_PER_W,), jnp.int32),
            pltpu.VMEM((N_TOKENS, DIM), jnp.float32),
        ] + [pltpu.VMEM((CHUNK, DIM), jnp.float32)] * NBUF
          + [pltpu.SemaphoreType.DMA] * (2 * NBUF),
    )
    return run(table, idx_flat, bias)


@jax.jit
def kernel(x_cat, tables, id_emb, type_emb):
    bias = _make_bias(id_emb, type_emb)
    out = _sc_gather(tables.reshape(N_TOKENS * VOCAB, DIM),
                     x_cat.T.reshape(ROWS), bias)
    return out.reshape(N_TOKENS, B, DIM).transpose(1, 0, 2)


# speedup vs baseline: 1.0061x; 1.0061x over previous
import jax
import jax.numpy as jnp
from jax import lax
from jax.experimental import pallas as pl
from jax.experimental.pallas import tpu as pltpu
from jax.experimental.pallas import tpu_sc as plsc

B = 16384
N_TOKENS = 43
VOCAB = 1000
DIM = 128

ROWS = B * N_TOKENS
NW = 32
CHUNK = 128
ROWS_PER_W = ROWS // NW
CHUNKS_PER_W = ROWS_PER_W // CHUNK


def _bias_body(id_ref, type_ref, out_ref):
    out_ref[...] = id_ref[0] + type_ref[0]


def _make_bias(id_emb, type_emb):
    return pl.pallas_call(
        _bias_body,
        out_shape=jax.ShapeDtypeStruct((N_TOKENS, DIM), jnp.float32),
    )(id_emb, type_emb)


NBUF = 2
PAIRS_PER_W = CHUNKS_PER_W // 2
GROUPS = PAIRS_PER_W // NBUF


def _gather_body(table_hbm, idx_hbm, bias_hbm, out_hbm, idx_v, bias_v,
                 *scratch):
    bufs = scratch[:NBUF]
    gsem = scratch[NBUF:2 * NBUF]
    osem = scratch[2 * NBUF:]
    wid = lax.axis_index("s") * 2 + lax.axis_index("c")
    row_base = wid * ROWS_PER_W
    pltpu.sync_copy(idx_hbm.at[pl.ds(row_base, ROWS_PER_W)], idx_v)
    pltpu.sync_copy(bias_hbm, bias_v)

    def chunk_t(j):
        return (row_base + j * CHUNK) // B

    def tbl_at(j):
        return table_hbm.at[pl.ds(chunk_t(j) * VOCAB, VOCAB)]

    def idx_at(j):
        return idx_v.at[pl.ds(j * CHUNK, CHUNK)]

    def add_bias(j, buf, base):
        t = chunk_t(j)
        bv = [bias_v[t, pl.ds(16 * k, 16)] for k in range(DIM // 16)]

        def row_add(r, carry):
            for u in range(4):
                for k in range(DIM // 16):
                    sl = pl.ds(16 * k, 16)
                    buf[r + u, sl] = buf[r + u, sl] + bv[k]
            return carry

        lax.fori_loop(0, CHUNK // 4, lambda i, c: row_add(base + 4 * i, c), 0,
                      unroll=False)

    def gather_pair(p, s):
        pltpu.async_copy(tbl_at(2 * p).at[idx_at(2 * p)],
                         bufs[s].at[pl.ds(0, CHUNK)], gsem[s])
        pltpu.async_copy(tbl_at(2 * p + 1).at[idx_at(2 * p + 1)],
                         bufs[s].at[pl.ds(CHUNK, CHUNK)], gsem[s])

    def wait_pair(p, s):
        pltpu.make_async_copy(tbl_at(2 * p).at[idx_at(2 * p)],
                              bufs[s].at[pl.ds(0, CHUNK)], gsem[s]).wait()
        pltpu.make_async_copy(tbl_at(2 * p + 1).at[idx_at(2 * p + 1)],
                              bufs[s].at[pl.ds(CHUNK, CHUNK)], gsem[s]).wait()

    def out_at(p):
        return out_hbm.at[pl.ds(row_base + p * 2 * CHUNK, 2 * CHUNK)]

    for s in range(NBUF):
        gather_pair(s, s)

    def step(i, carry):
        for s in range(NBUF):
            p = i * NBUF + s
            wait_pair(p, s)
            add_bias(2 * p, bufs[s], 0)
            add_bias(2 * p + 1, bufs[s], CHUNK)
            pltpu.async_copy(bufs[s], out_at(p), osem[s])
        for s in range(NBUF):
            p = i * NBUF + s
            pltpu.make_async_copy(bufs[s], out_at(p), osem[s]).wait()
            gather_pair(p + NBUF, s)
        return carry

    lax.fori_loop(0, GROUPS - 1, step, 0)

    for s in range(NBUF):
        p = (GROUPS - 1) * NBUF + s
        wait_pair(p, s)
        add_bias(2 * p, bufs[s], 0)
        add_bias(2 * p + 1, bufs[s], CHUNK)
        pltpu.async_copy(bufs[s], out_at(p), osem[s])
    for s in range(NBUF):
        p = (GROUPS - 1) * NBUF + s
        pltpu.make_async_copy(bufs[s], out_at(p), osem[s]).wait()


def _sc_gather(table, idx_flat, bias):
    mesh = plsc.VectorSubcoreMesh(core_axis_name="c", subcore_axis_name="s")
    run = pl.kernel(
        _gather_body,
        out_type=jax.ShapeDtypeStruct((ROWS, DIM), jnp.float32),
        mesh=mesh,
        scratch_types=[
            pltpu.VMEM((ROWS_PER_W,), jnp.int32),
            pltpu.VMEM((N_TOKENS, DIM), jnp.float32),
        ] + [pltpu.VMEM((2 * CHUNK, DIM), jnp.float32)] * NBUF
          + [pltpu.SemaphoreType.DMA] * (2 * NBUF),
    )
    return run(table, idx_flat, bias)


@jax.jit
def kernel(x_cat, tables, id_emb, type_emb):
    bias = _make_bias(id_emb, type_emb)
    out = _sc_gather(tables.reshape(N_TOKENS * VOCAB, DIM),
                     x_cat.T.reshape(ROWS), bias)
    return out.reshape(N_TOKENS, B, DIM).transpose(1, 0, 2)
